# Initial kernel scaffold; baseline (speedup 1.0000x reference)
import functools

import jax
import jax.numpy as jnp
from jax import lax
from jax.experimental import pallas as pl
from jax.experimental.pallas import tpu as pltpu
from jax.experimental.pallas import tpu_sc as plsc

ALPHA = 0.1
K = 5
N = 10000
E = 320000
D = 128
DH = 64
NS = 16
LANES = 16
EROW = 128
NEROWS = E // EROW
ROWS_PER_TILE = NEROWS // NS
NODE_ROWS = N // NS
RB = 125
NRB = NODE_ROWS // RB


def _scale_group(msg, wrow, g):
  for e in range(16):
    lin = g * 16 + e
    wb = plsc.load_gather(wrow, [jnp.full((LANES,), lin, jnp.int32)])
    for q in range(DH // LANES):
      sl = pl.ds(q * LANES, LANES)
      msg[lin, sl] = msg[lin, sl] * wb


def _body(h0f, srcb, dst2, w2, svec, out, hwork, idx_s, idx_d, wrow, msg,
          rbuf_a, rbuf_h, zbuf, svec_v, acc_sh, sem):
  c = lax.axis_index("c")
  s = lax.axis_index("s")
  node0 = s * NODE_ROWS
  cbase = c * N

  def zinit(r, _):
    for q in range(DH // LANES):
      zbuf[r, pl.ds(q * LANES, LANES)] = jnp.zeros((LANES,), jnp.float32)
    return 0
  lax.fori_loop(0, RB, zinit, 0)

  for b in range(NRB):
    r0 = node0 + b * RB
    pltpu.sync_copy(zbuf, acc_sh.at[pl.ds(r0, RB)])
    pltpu.sync_copy(h0f.at[pl.ds(cbase + r0, RB)], rbuf_h)
    pltpu.sync_copy(rbuf_h, hwork.at[pl.ds(cbase + r0, RB)])

  pltpu.sync_copy(svec, svec_v)
  sv = svec_v[...]
  tv = 1.0 - 2.0 / (jnp.exp(2.0 * sv) + 1.0)

  plsc.subcore_barrier()

  def do_edge_row(r):
    pltpu.sync_copy(srcb.at[c, r], idx_s)
    pltpu.sync_copy(dst2.at[r], idx_d)
    pltpu.sync_copy(w2.at[r], wrow)
    pltpu.async_copy(hwork.at[idx_s], msg, sem).wait()
    def grp(g, _):
      _scale_group(msg, wrow, g)
      return 0
    lax.fori_loop(0, EROW // 16, grp, 0)
    pltpu.sync_copy(msg, acc_sh.at[idx_d], add=True)

  def step(k, _):
    def edge_iter(i, _):
      do_edge_row(s * ROWS_PER_TILE + i)
      return 0
    lax.fori_loop(0, ROWS_PER_TILE, edge_iter, 0)

    @pl.when(s < NEROWS - NS * ROWS_PER_TILE)
    def _extra():
      do_edge_row(NS * ROWS_PER_TILE + s)

    plsc.subcore_barrier()

    for b in range(NRB):
      r0 = node0 + b * RB
      pltpu.sync_copy(acc_sh.at[pl.ds(r0, RB)], rbuf_a)
      pltpu.sync_copy(h0f.at[pl.ds(cbase + r0, RB)], rbuf_h)
      pltpu.sync_copy(zbuf, acc_sh.at[pl.ds(r0, RB)])
      def blend(r, _):
        for q in range(DH // LANES):
          sl = pl.ds(q * LANES, LANES)
          rbuf_a[r, sl] = (1.0 - ALPHA) * rbuf_a[r, sl] + ALPHA * rbuf_h[r, sl]
        return 0
      lax.fori_loop(0, RB, blend, 0)
      pltpu.sync_copy(rbuf_a, hwork.at[pl.ds(cbase + r0, RB)])

    plsc.subcore_barrier()
    return 0

  lax.fori_loop(0, K, step, 0)

  for b in range(NRB):
    r0 = node0 + b * RB
    pltpu.sync_copy(hwork.at[pl.ds(cbase + r0, RB)], rbuf_a)
    pltpu.sync_copy(h0f.at[pl.ds(cbase + r0, RB)], rbuf_h)
    def final(r, _):
      for q in range(DH // LANES):
        sl = pl.ds(q * LANES, LANES)
        rbuf_a[r, sl] = rbuf_h[r, sl] + tv * rbuf_a[r, sl]
      return 0
    lax.fori_loop(0, RB, final, 0)
    pltpu.sync_copy(rbuf_a, out.at[pl.ds(cbase + r0, RB)])


@jax.jit
def _run(h0f, srcb, dst2, w2, svec):
  mesh = plsc.VectorSubcoreMesh(core_axis_name="c", subcore_axis_name="s")
  f = pl.kernel(
      _body,
      out_type=(
          jax.ShapeDtypeStruct((2 * N, DH), jnp.float32),
          jax.ShapeDtypeStruct((2 * N, DH), jnp.float32),
      ),
      mesh=mesh,
      scratch_types=[
          pltpu.VMEM((EROW,), jnp.int32),
          pltpu.VMEM((EROW,), jnp.int32),
          pltpu.VMEM((EROW,), jnp.float32),
          pltpu.VMEM((EROW, DH), jnp.float32),
          pltpu.VMEM((RB, DH), jnp.float32),
          pltpu.VMEM((RB, DH), jnp.float32),
          pltpu.VMEM((RB, DH), jnp.float32),
          pltpu.VMEM((LANES,), jnp.float32),
          pltpu.VMEM_SHARED((N, DH), jnp.float32),
          pltpu.SemaphoreType.DMA,
      ],
  )
  return f(h0f, srcb, dst2, w2, svec)


def kernel(h0, edge_index, edge_weight_norm, scale):
  src = edge_index[0].astype(jnp.int32).reshape(NEROWS, EROW)
  dst = edge_index[1].astype(jnp.int32).reshape(NEROWS, EROW)
  srcb = jnp.stack([src, src + N])
  w2 = edge_weight_norm.astype(jnp.float32).reshape(NEROWS, EROW)
  h0f = jnp.transpose(h0.reshape(N, 2, DH), (1, 0, 2)).reshape(2 * N, DH)
  svec = jnp.full((LANES,), scale, jnp.float32)
  outf, _ = _run(h0f, srcb, dst2=dst, w2=w2, svec=svec)
  return jnp.transpose(outf.reshape(2, N, DH), (1, 0, 2)).reshape(N, D)


# speedup vs baseline: 2.1632x; 2.1632x over previous
import jax
import jax.numpy as jnp
from jax import lax
from jax.experimental import pallas as pl
from jax.experimental.pallas import tpu as pltpu
from jax.experimental.pallas import tpu_sc as plsc

ALPHA = 0.1
K = 5
N = 10000
NPAD = 10240
E = 320000
D = 128
DH = 64
NS = 16
LANES = 16
EROW = 128
NEROWS = E // EROW
ROWS_PER_TILE = NEROWS // NS
NODE_ROWS = NPAD // NS
RB = 128
NRB = NODE_ROWS // RB


_GATHER_DNUMS = lax.GatherDimensionNumbers(
    offset_dims=(), collapsed_slice_dims=(0,), start_index_map=(0,))


def _lane_broadcast(vec, e):
  idx = jnp.full((LANES, 1), e, jnp.int32)
  return lax.gather(vec, idx, _GATHER_DNUMS, (1,),
                    mode=lax.GatherScatterMode.PROMISE_IN_BOUNDS)


def _scale_group(msg, wrow, g):
  wv = wrow[pl.ds(g * LANES, LANES)]
  for e in range(16):
    lin = g * 16 + e
    wb = _lane_broadcast(wv, e)
    for q in range(DH // LANES):
      sl = pl.ds(q * LANES, LANES)
      msg[lin, sl] = msg[lin, sl] * wb


def _body(h0f, srcc, dstc, wc, svec, out, hwork, idx_s, idx_d, wrow, msg,
          rbuf_a, rbuf_h, zbuf, svec_v, acc_sh, sem):
  c = lax.axis_index("c")
  s = lax.axis_index("s")
  node0 = pl.multiple_of(s * NODE_ROWS, RB)
  cbase = pl.multiple_of(c * NPAD, RB)
  ebase = pl.multiple_of(c * E, EROW)

  def zinit(r, _):
    for q in range(DH // LANES):
      zbuf[r, pl.ds(q * LANES, LANES)] = jnp.zeros((LANES,), jnp.float32)
    return 0
  lax.fori_loop(0, RB, zinit, 0)

  for b in range(NRB):
    r0 = node0 + b * RB
    pltpu.sync_copy(zbuf, acc_sh.at[pl.ds(r0, RB)])
    pltpu.sync_copy(h0f.at[pl.ds(cbase + r0, RB)], rbuf_h)
    pltpu.sync_copy(rbuf_h, hwork.at[pl.ds(cbase + r0, RB)])

  pltpu.sync_copy(svec, svec_v)
  sv = svec_v[...]
  tv = 1.0 - 2.0 / (jnp.exp(2.0 * sv) + 1.0)

  plsc.subcore_barrier()

  def do_edge_row(r):
    e0 = pl.multiple_of(r * EROW, EROW)
    pltpu.sync_copy(srcc.at[pl.ds(ebase + e0, EROW)], idx_s)
    pltpu.sync_copy(dstc.at[pl.ds(e0, EROW)], idx_d)
    pltpu.sync_copy(wc.at[pl.ds(e0, EROW)], wrow)
    pltpu.async_copy(hwork.at[idx_s], msg, sem).wait()
    def grp(g, _):
      _scale_group(msg, wrow, g)
      return 0
    lax.fori_loop(0, EROW // 16, grp, 0)
    pltpu.sync_copy(msg, acc_sh.at[idx_d], add=True)

  def step(k, _):
    def edge_iter(i, _):
      do_edge_row(s * ROWS_PER_TILE + i)
      return 0
    lax.fori_loop(0, ROWS_PER_TILE, edge_iter, 0)

    @pl.when(s < NEROWS - NS * ROWS_PER_TILE)
    def _extra():
      do_edge_row(NS * ROWS_PER_TILE + s)

    plsc.subcore_barrier()

    for b in range(NRB):
      r0 = node0 + b * RB
      pltpu.sync_copy(acc_sh.at[pl.ds(r0, RB)], rbuf_a)
      pltpu.sync_copy(h0f.at[pl.ds(cbase + r0, RB)], rbuf_h)
      pltpu.sync_copy(zbuf, acc_sh.at[pl.ds(r0, RB)])
      def blend(r, _):
        for q in range(DH // LANES):
          sl = pl.ds(q * LANES, LANES)
          rbuf_a[r, sl] = (1.0 - ALPHA) * rbuf_a[r, sl] + ALPHA * rbuf_h[r, sl]
        return 0
      lax.fori_loop(0, RB, blend, 0)
      pltpu.sync_copy(rbuf_a, hwork.at[pl.ds(cbase + r0, RB)])

    plsc.subcore_barrier()
    return 0

  lax.fori_loop(0, K, step, 0)

  for b in range(NRB):
    r0 = node0 + b * RB
    pltpu.sync_copy(hwork.at[pl.ds(cbase + r0, RB)], rbuf_a)
    pltpu.sync_copy(h0f.at[pl.ds(cbase + r0, RB)], rbuf_h)
    def final(r, _):
      for q in range(DH // LANES):
        sl = pl.ds(q * LANES, LANES)
        rbuf_a[r, sl] = rbuf_h[r, sl] + tv * rbuf_a[r, sl]
      return 0
    lax.fori_loop(0, RB, final, 0)
    pltpu.sync_copy(rbuf_a, out.at[pl.ds(cbase + r0, RB)])


@jax.jit
def _run(h0f, srcc, dstc, wc, svec):
  mesh = plsc.VectorSubcoreMesh(core_axis_name="c", subcore_axis_name="s")
  f = pl.kernel(
      _body,
      out_type=(
          jax.ShapeDtypeStruct((2 * NPAD, DH), jnp.float32),
          jax.ShapeDtypeStruct((2 * NPAD, DH), jnp.float32),
      ),
      mesh=mesh,
      compiler_params=pltpu.CompilerParams(use_tc_tiling_on_sc=False),
      scratch_types=[
          pltpu.VMEM((EROW,), jnp.int32),
          pltpu.VMEM((EROW,), jnp.int32),
          pltpu.VMEM((EROW,), jnp.float32),
          pltpu.VMEM((EROW, DH), jnp.float32),
          pltpu.VMEM((RB, DH), jnp.float32),
          pltpu.VMEM((RB, DH), jnp.float32),
          pltpu.VMEM((RB, DH), jnp.float32),
          pltpu.VMEM((LANES,), jnp.float32),
          pltpu.VMEM_SHARED((NPAD, DH), jnp.float32),
          pltpu.SemaphoreType.DMA,
      ],
  )
  return f(h0f, srcc, dstc, wc, svec)


def kernel(h0, edge_index, edge_weight_norm, scale):
  src = edge_index[0].astype(jnp.int32)
  dst = edge_index[1].astype(jnp.int32)
  srcc = jnp.concatenate([src, src + NPAD])
  wc = edge_weight_norm.astype(jnp.float32)
  h0r = jnp.transpose(h0.reshape(N, 2, DH), (1, 0, 2))
  h0f = jnp.pad(h0r, ((0, 0), (0, NPAD - N), (0, 0))).reshape(2 * NPAD, DH)
  svec = jnp.full((LANES,), scale, jnp.float32)
  outf, _ = _run(h0f, srcc, dst, wc, svec)
  outr = outf.reshape(2, NPAD, DH)[:, :N]
  return jnp.transpose(outr, (1, 0, 2)).reshape(N, D)
